# Initial kernel scaffold; baseline (speedup 1.0000x reference)
import functools

import jax
import jax.numpy as jnp
from jax import lax
from jax.experimental import pallas as pl
from jax.experimental.pallas import tpu as pltpu
from jax.experimental.pallas import tpu_sc as plsc

_N = 10000
_E = 640000
_H = 128
_NC = 2
_NS = 16
_NW = _NC * _NS
_EPW = _E // _NW
_CK = 80
_NCHUNK = _EPW // _CK
_NPAD = 10240
_RPT = _NPAD // _NS

_mesh = plsc.VectorSubcoreMesh(
    core_axis_name="c", subcore_axis_name="s", num_cores=_NC, num_subcores=_NS
)


@functools.partial(
    pl.kernel,
    out_type=jax.ShapeDtypeStruct((_NC, _NPAD), jnp.float32),
    mesh=_mesh,
    scratch_types=[
        pltpu.VMEM((_CK,), jnp.int32),
        pltpu.VMEM((_CK,), jnp.float32),
        pltpu.VMEM((_RPT,), jnp.float32),
        pltpu.VMEM_SHARED((_NPAD,), jnp.float32),
    ],
)
def _sc_deg(dst_hbm, ew_hbm, out_hbm, idx_v, ew_v, zed_v, acc_sh):
    c = lax.axis_index("c")
    s = lax.axis_index("s")
    wid = s * _NC + c

    def _z(i, carry):
        zed_v[pl.ds(i * 16, 16)] = jnp.zeros((16,), jnp.float32)
        return carry

    lax.fori_loop(0, _RPT // 16, _z, 0)
    pltpu.sync_copy(zed_v, acc_sh.at[pl.ds(s * _RPT, _RPT)])
    plsc.subcore_barrier()

    def _chunk(i, carry):
        base = wid * _EPW + i * _CK
        pltpu.sync_copy(dst_hbm.at[pl.ds(base, _CK)], idx_v)
        pltpu.sync_copy(ew_hbm.at[pl.ds(base, _CK)], ew_v)
        pltpu.sync_copy(ew_v, acc_sh.at[idx_v], add=True)
        return carry

    lax.fori_loop(0, _NCHUNK, _chunk, 0)
    plsc.subcore_barrier()
    pltpu.sync_copy(acc_sh.at[pl.ds(s * _RPT, _RPT)], out_hbm.at[c, pl.ds(s * _RPT, _RPT)])


@functools.partial(
    pl.kernel,
    out_type=jax.ShapeDtypeStruct((_NC, _NPAD, _H), jnp.float32),
    mesh=_mesh,
    scratch_types=[
        pltpu.VMEM((_CK,), jnp.int32),
        pltpu.VMEM((_CK,), jnp.float32),
        pltpu.VMEM((_CK, _H), jnp.float32),
        pltpu.VMEM((128, _H), jnp.float32),
        pltpu.VMEM_SHARED((_NPAD, _H), jnp.float32),
        pltpu.SemaphoreType.DMA,
    ],
)
def _sc_scatter(y_hbm, src_hbm, dst_hbm, ew_hbm, out_hbm, idx_v, ew_v, rows_v, zed_v, acc_sh, sem):
    c = lax.axis_index("c")
    s = lax.axis_index("s")
    wid = s * _NC + c

    def _zrow(i, carry):
        for p in range(_H // 16):
            zed_v[i, pl.ds(p * 16, 16)] = jnp.zeros((16,), jnp.float32)
        return carry

    lax.fori_loop(0, 128, _zrow, 0)
    for k in range(_RPT // 128):
        pltpu.sync_copy(zed_v, acc_sh.at[pl.ds(s * _RPT + k * 128, 128)])
    plsc.subcore_barrier()

    def _chunk(i, carry):
        base = wid * _EPW + i * _CK
        pltpu.sync_copy(src_hbm.at[pl.ds(base, _CK)], idx_v)
        pltpu.sync_copy(ew_hbm.at[pl.ds(base, _CK)], ew_v)
        pltpu.async_copy(y_hbm.at[idx_v], rows_v, sem).wait()

        def _srow(j, inner):
            w = plsc.load_gather(ew_v, [jnp.full((16,), 0, jnp.int32) + j])
            for p in range(_H // 16):
                rows_v[j, pl.ds(p * 16, 16)] = rows_v[j, pl.ds(p * 16, 16)] * w
            return inner

        lax.fori_loop(0, _CK, _srow, 0)
        pltpu.sync_copy(dst_hbm.at[pl.ds(base, _CK)], idx_v)
        pltpu.sync_copy(rows_v, acc_sh.at[idx_v], add=True)
        return carry

    lax.fori_loop(0, _NCHUNK, _chunk, 0)
    plsc.subcore_barrier()
    pltpu.sync_copy(acc_sh.at[pl.ds(s * _RPT, _RPT)], out_hbm.at[c, pl.ds(s * _RPT, _RPT)])


def _sigmoid(v):
    return 1.0 / (1.0 + jnp.exp(-v))


def _tc_prep_body(degp_ref, dinv_ref):
    deg = degp_ref[0] + degp_ref[1] + 1.0
    safe = jnp.where(deg > 0, deg, 1.0)
    dinv_ref[...] = jnp.where(deg > 0, 1.0 / jnp.sqrt(safe), 0.0)


_tc_prep = pl.pallas_call(
    _tc_prep_body, out_shape=jax.ShapeDtypeStruct((_NPAD,), jnp.float32)
)


def _tc_y0_body(x_ref, w_ref, dinv_ref, y_ref):
    y_ref[...] = (
        jnp.dot(x_ref[...], w_ref[...], preferred_element_type=jnp.float32)
        * dinv_ref[...]
    )


_tc_y0 = pl.pallas_call(
    _tc_y0_body, out_shape=jax.ShapeDtypeStruct((_N, _H), jnp.float32)
)


def _bn_silu(p_ref, y_ref, dinv_ref, b_ref, g_ref, be_ref):
    pp = p_ref[0, :_N, :] + p_ref[1, :_N, :]
    dinv = dinv_ref[...]
    agg = dinv * (pp + y_ref[...]) + b_ref[...]
    mu = jnp.mean(agg, axis=0, keepdims=True)
    var = jnp.mean((agg - mu) ** 2, axis=0, keepdims=True)
    hn = (agg - mu) / jnp.sqrt(var + 1e-5) * g_ref[...] + be_ref[...]
    return hn * _sigmoid(hn)


def _tc_layer_body(p_ref, y_ref, dinv_ref, b_ref, g_ref, be_ref, w_ref, o_ref):
    h = _bn_silu(p_ref, y_ref, dinv_ref, b_ref, g_ref, be_ref)
    o_ref[...] = (
        jnp.dot(h, w_ref[...], preferred_element_type=jnp.float32) * dinv_ref[...]
    )


_tc_layer = pl.pallas_call(
    _tc_layer_body, out_shape=jax.ShapeDtypeStruct((_N, _H), jnp.float32)
)


def _tc_final_body(p_ref, y_ref, dinv_ref, b_ref, g_ref, be_ref,
                   fc1w_ref, fc1b_ref, fc2w_ref, fc2b_ref, o_ref):
    h = _bn_silu(p_ref, y_ref, dinv_ref, b_ref, g_ref, be_ref)
    pooled = jnp.mean(h, axis=0, keepdims=True)
    o1 = jnp.dot(pooled, fc1w_ref[...], preferred_element_type=jnp.float32) + fc1b_ref[...]
    o1 = o1 * _sigmoid(o1)
    o2 = jnp.dot(o1, fc2w_ref[...], preferred_element_type=jnp.float32) + fc2b_ref[...]
    o_ref[...] = _sigmoid(o2)


_tc_final = pl.pallas_call(
    _tc_final_body, out_shape=jax.ShapeDtypeStruct((1, 1), jnp.float32)
)


def kernel(x, edge_index, edge_attr, batch, W0, b0, g0, be0, W1, b1, g1, be1,
           W2, b2, g2, be2, W3, b3, g3, be3, fc1_W, fc1_b, fc2_W, fc2_b):
    src = edge_index[0]
    dst = edge_index[1]
    ew = edge_attr.reshape(-1)

    degp = _sc_deg(dst, ew)
    dinv_full = _tc_prep(degp)
    dinv_col = dinv_full[:_N, None]

    bs = (b0, b1, b2, b3)
    gs = (g0, g1, g2, g3)
    bes = (be0, be1, be2, be3)
    Ws = (W0, W1, W2, W3)

    y = _tc_y0(x, W0, dinv_col)
    out = None
    for i in range(4):
        parts = _sc_scatter(y, src, dst, ew)
        if i < 3:
            y = _tc_layer(parts, y, dinv_col, bs[i], gs[i], bes[i], Ws[i + 1])
        else:
            out = _tc_final(parts, y, dinv_col, bs[3], gs[3], bes[3],
                            fc1_W, fc1_b, fc2_W, fc2_b)
    return out


# speedup vs baseline: 7.9231x; 7.9231x over previous
import functools

import jax
import jax.numpy as jnp
from jax import lax
from jax.experimental import pallas as pl
from jax.experimental.pallas import tpu as pltpu
from jax.experimental.pallas import tpu_sc as plsc

_N = 10000
_E = 640000
_H = 128
_NC = 2
_NS = 16
_NW = _NC * _NS
_EPW = _E // _NW
_CK = 80
_NCHUNK = _EPW // _CK
_NPAD = 10240
_RPT = _NPAD // _NS

_mesh = plsc.VectorSubcoreMesh(
    core_axis_name="c", subcore_axis_name="s", num_cores=_NC, num_subcores=_NS
)


@functools.partial(
    pl.kernel,
    out_type=jax.ShapeDtypeStruct((_NC, _NPAD), jnp.float32),
    mesh=_mesh,
    scratch_types=[
        pltpu.VMEM((_CK,), jnp.int32),
        pltpu.VMEM((_CK,), jnp.float32),
        pltpu.VMEM((_RPT,), jnp.float32),
        pltpu.VMEM_SHARED((_NPAD,), jnp.float32),
    ],
)
def _sc_deg(dst_hbm, ew_hbm, out_hbm, idx_v, ew_v, zed_v, acc_sh):
    c = lax.axis_index("c")
    s = lax.axis_index("s")
    wid = s * _NC + c

    def _z(i, carry):
        zed_v[pl.ds(i * 16, 16)] = jnp.zeros((16,), jnp.float32)
        return carry

    lax.fori_loop(0, _RPT // 16, _z, 0)
    pltpu.sync_copy(zed_v, acc_sh.at[pl.ds(s * _RPT, _RPT)])
    plsc.subcore_barrier()

    def _chunk(i, carry):
        base = wid * _EPW + i * _CK
        pltpu.sync_copy(dst_hbm.at[pl.ds(base, _CK)], idx_v)
        pltpu.sync_copy(ew_hbm.at[pl.ds(base, _CK)], ew_v)
        pltpu.sync_copy(ew_v, acc_sh.at[idx_v], add=True)
        return carry

    lax.fori_loop(0, _NCHUNK, _chunk, 0)
    plsc.subcore_barrier()
    pltpu.sync_copy(acc_sh.at[pl.ds(s * _RPT, _RPT)], out_hbm.at[c, pl.ds(s * _RPT, _RPT)])


@functools.partial(
    pl.kernel,
    out_type=jax.ShapeDtypeStruct((_NC, _NPAD, _H), jnp.float32),
    mesh=_mesh,
    scratch_types=[
        pltpu.VMEM((_CK,), jnp.int32),
        pltpu.VMEM((_CK,), jnp.float32),
        pltpu.VMEM((_CK, _H), jnp.float32),
        pltpu.VMEM((128, _H), jnp.float32),
        pltpu.VMEM_SHARED((_NPAD, _H), jnp.float32),
        pltpu.SemaphoreType.DMA,
    ],
)
def _sc_scatter(y_hbm, src_hbm, dst_hbm, ew_hbm, out_hbm, idx_v, ew_v, rows_v, zed_v, acc_sh, sem):
    c = lax.axis_index("c")
    s = lax.axis_index("s")
    wid = s * _NC + c

    def _zrow(i, carry):
        for p in range(_H // 16):
            zed_v[i, pl.ds(p * 16, 16)] = jnp.zeros((16,), jnp.float32)
        return carry

    lax.fori_loop(0, 128, _zrow, 0)
    for k in range(_RPT // 128):
        pltpu.sync_copy(zed_v, acc_sh.at[pl.ds(s * _RPT + k * 128, 128)])
    plsc.subcore_barrier()

    def _chunk(i, carry):
        base = wid * _EPW + i * _CK
        pltpu.sync_copy(src_hbm.at[pl.ds(base, _CK)], idx_v)
        pltpu.sync_copy(ew_hbm.at[pl.ds(base, _CK)], ew_v)
        pltpu.async_copy(y_hbm.at[idx_v], rows_v, sem).wait()

        def _sgrp(g, inner):
            ew16 = ew_v[pl.ds(g * 16, 16)]
            for jj in range(16):
                j = g * 16 + jj
                w = ew16[jj]
                for p in range(_H // 16):
                    rows_v[j, pl.ds(p * 16, 16)] = rows_v[j, pl.ds(p * 16, 16)] * w
            return inner

        lax.fori_loop(0, _CK // 16, _sgrp, 0)
        pltpu.sync_copy(dst_hbm.at[pl.ds(base, _CK)], idx_v)
        pltpu.sync_copy(rows_v, acc_sh.at[idx_v], add=True)
        return carry

    lax.fori_loop(0, _NCHUNK, _chunk, 0)
    plsc.subcore_barrier()
    pltpu.sync_copy(acc_sh.at[pl.ds(s * _RPT, _RPT)], out_hbm.at[c, pl.ds(s * _RPT, _RPT)])


def _sigmoid(v):
    return 1.0 / (1.0 + jnp.exp(-v))


def _tc_prep_body(degp_ref, dinv_ref):
    deg = degp_ref[0] + degp_ref[1] + 1.0
    safe = jnp.where(deg > 0, deg, 1.0)
    dinv_ref[...] = jnp.where(deg > 0, 1.0 / jnp.sqrt(safe), 0.0)


_tc_prep = pl.pallas_call(
    _tc_prep_body, out_shape=jax.ShapeDtypeStruct((_NPAD,), jnp.float32)
)


def _tc_y0_body(x_ref, w_ref, dinv_ref, y_ref):
    y_ref[...] = (
        jnp.dot(x_ref[...], w_ref[...], preferred_element_type=jnp.float32)
        * dinv_ref[...]
    )


_tc_y0 = pl.pallas_call(
    _tc_y0_body, out_shape=jax.ShapeDtypeStruct((_N, _H), jnp.float32)
)


def _bn_silu(p_ref, y_ref, dinv_ref, b_ref, g_ref, be_ref):
    pp = p_ref[0, :_N, :] + p_ref[1, :_N, :]
    dinv = dinv_ref[...]
    agg = dinv * (pp + y_ref[...]) + b_ref[...]
    mu = jnp.mean(agg, axis=0, keepdims=True)
    var = jnp.mean((agg - mu) ** 2, axis=0, keepdims=True)
    hn = (agg - mu) / jnp.sqrt(var + 1e-5) * g_ref[...] + be_ref[...]
    return hn * _sigmoid(hn)


def _tc_layer_body(p_ref, y_ref, dinv_ref, b_ref, g_ref, be_ref, w_ref, o_ref):
    h = _bn_silu(p_ref, y_ref, dinv_ref, b_ref, g_ref, be_ref)
    o_ref[...] = (
        jnp.dot(h, w_ref[...], preferred_element_type=jnp.float32) * dinv_ref[...]
    )


_tc_layer = pl.pallas_call(
    _tc_layer_body, out_shape=jax.ShapeDtypeStruct((_N, _H), jnp.float32)
)


def _tc_final_body(p_ref, y_ref, dinv_ref, b_ref, g_ref, be_ref,
                   fc1w_ref, fc1b_ref, fc2w_ref, fc2b_ref, o_ref):
    h = _bn_silu(p_ref, y_ref, dinv_ref, b_ref, g_ref, be_ref)
    pooled = jnp.mean(h, axis=0, keepdims=True)
    o1 = jnp.dot(pooled, fc1w_ref[...], preferred_element_type=jnp.float32) + fc1b_ref[...]
    o1 = o1 * _sigmoid(o1)
    o2 = jnp.dot(o1, fc2w_ref[...], preferred_element_type=jnp.float32) + fc2b_ref[...]
    o_ref[...] = _sigmoid(o2)


_tc_final = pl.pallas_call(
    _tc_final_body, out_shape=jax.ShapeDtypeStruct((1, 1), jnp.float32)
)


def kernel(x, edge_index, edge_attr, batch, W0, b0, g0, be0, W1, b1, g1, be1,
           W2, b2, g2, be2, W3, b3, g3, be3, fc1_W, fc1_b, fc2_W, fc2_b):
    src = edge_index[0]
    dst = edge_index[1]
    ew = edge_attr.reshape(-1)

    degp = _sc_deg(dst, ew)
    dinv_full = _tc_prep(degp)
    dinv_col = dinv_full[:_N, None]

    bs = (b0, b1, b2, b3)
    gs = (g0, g1, g2, g3)
    bes = (be0, be1, be2, be3)
    Ws = (W0, W1, W2, W3)

    y = _tc_y0(x, W0, dinv_col)
    out = None
    for i in range(4):
        parts = _sc_scatter(y, src, dst, ew)
        if i < 3:
            y = _tc_layer(parts, y, dinv_col, bs[i], gs[i], bes[i], Ws[i + 1])
        else:
            out = _tc_final(parts, y, dinv_col, bs[3], gs[3], bes[3],
                            fc1_W, fc1_b, fc2_W, fc2_b)
    return out


# speedup vs baseline: 11.3330x; 1.4304x over previous
import functools

import jax
import jax.numpy as jnp
from jax import lax
from jax.experimental import pallas as pl
from jax.experimental.pallas import tpu as pltpu
from jax.experimental.pallas import tpu_sc as plsc

_N = 10000
_E = 640000
_H = 128
_NC = 2
_NS = 16
_NW = _NC * _NS
_EPW = _E // _NW
_CK = 80
_NCHUNK = _EPW // _CK
_NPAD = 10240
_RPT = _NPAD // _NS
_SG = 5
_NSG = _NCHUNK // _SG
_DRING = 5

_mesh = plsc.VectorSubcoreMesh(
    core_axis_name="c", subcore_axis_name="s", num_cores=_NC, num_subcores=_NS
)


@functools.partial(
    pl.kernel,
    out_type=(jax.ShapeDtypeStruct((_NPAD,), jnp.float32),
              jax.ShapeDtypeStruct((_NPAD,), jnp.float32)),
    mesh=_mesh,
    scratch_types=[
        pltpu.VMEM((_CK,), jnp.int32),
        pltpu.VMEM((_CK,), jnp.float32),
        pltpu.VMEM((_RPT,), jnp.float32),
        pltpu.VMEM_SHARED((_NPAD,), jnp.float32),
    ],
)
def _sc_deg(dst_hbm, ew_hbm, out0_hbm, out1_hbm, idx_v, ew_v, zed_v, acc_sh):
    c = lax.axis_index("c")
    s = lax.axis_index("s")
    wid = s * _NC + c

    def _z(i, carry):
        zed_v[pl.ds(i * 16, 16)] = jnp.zeros((16,), jnp.float32)
        return carry

    lax.fori_loop(0, _RPT // 16, _z, 0)
    pltpu.sync_copy(zed_v, acc_sh.at[pl.ds(s * _RPT, _RPT)])
    plsc.subcore_barrier()

    def _chunk(i, carry):
        base = wid * _EPW + i * _CK
        pltpu.sync_copy(dst_hbm.at[pl.ds(base, _CK)], idx_v)
        pltpu.sync_copy(ew_hbm.at[pl.ds(base, _CK)], ew_v)
        pltpu.sync_copy(ew_v, acc_sh.at[idx_v], add=True)
        return carry

    lax.fori_loop(0, _NCHUNK, _chunk, 0)
    plsc.subcore_barrier()

    @pl.when(c == 0)
    def _w0():
        pltpu.sync_copy(acc_sh.at[pl.ds(s * _RPT, _RPT)], out0_hbm.at[pl.ds(s * _RPT, _RPT)])

    @pl.when(c == 1)
    def _w1():
        pltpu.sync_copy(acc_sh.at[pl.ds(s * _RPT, _RPT)], out1_hbm.at[pl.ds(s * _RPT, _RPT)])


_CKS = 40
_NCHUNKS = _EPW // _CKS
_NSGS = _NCHUNKS // _SG
_NG16 = (_CKS + 15) // 16


@functools.partial(
    pl.kernel,
    out_type=jax.ShapeDtypeStruct((_NC, _NPAD, _H), jnp.float32),
    mesh=_mesh,
    scratch_types=[
        pltpu.VMEM((2 * _SG, 2, _CKS), jnp.int32),
        pltpu.VMEM((2 * _SG, _CKS), jnp.float32),
    ]
    + [pltpu.VMEM((_CKS, _H), jnp.float32)] * 4
    + [pltpu.VMEM_SHARED((_NPAD, _H), jnp.float32)]
    + [pltpu.SemaphoreType.DMA] * 6,
)
def _sc_scatter(y_hbm, rec_hbm, ewr_hbm, out_hbm, rec_v, ew_v,
                gbuf0, gbuf1, sbuf0, sbuf1, acc_sh, rsem, wsem,
                gsem0, gsem1, ssem0, ssem1):
    gbufs = (gbuf0, gbuf1)
    sbufs = (sbuf0, sbuf1)
    gsems = (gsem0, gsem1)
    ssems = (ssem0, ssem1)

    c = lax.axis_index("c")
    s = lax.axis_index("s")
    wid = s * _NC + c

    pltpu.async_copy(rec_hbm.at[wid, 0], rec_v.at[pl.ds(0, _SG)], rsem)
    pltpu.async_copy(ewr_hbm.at[wid, 0], ew_v.at[pl.ds(0, _SG)], wsem)

    def _zrow(j, carry):
        for p in range(_H // 16):
            sbuf0[j, pl.ds(p * 16, 16)] = jnp.zeros((16,), jnp.float32)
        return carry

    lax.fori_loop(0, _CKS, _zrow, 0)
    for k in range(_RPT // _CKS):
        pltpu.sync_copy(sbuf0, acc_sh.at[pl.ds(s * _RPT + k * _CKS, _CKS)])
    pltpu.sync_copy(
        sbuf0.at[pl.ds(0, _RPT % _CKS)],
        acc_sh.at[pl.ds(s * _RPT + (_RPT // _CKS) * _CKS, _RPT % _CKS)])
    plsc.subcore_barrier()

    def _sg_body(sg, carry):
        p = lax.rem(sg, 2)
        rowbase = p * _SG
        pltpu.make_async_copy(
            rec_hbm.at[wid, 0], rec_v.at[pl.ds(0, _SG)], rsem).wait()
        pltpu.make_async_copy(
            ewr_hbm.at[wid, 0], ew_v.at[pl.ds(0, _SG)], wsem).wait()
        for kk in range(2):
            pltpu.async_copy(
                y_hbm.at[rec_v.at[rowbase + kk, 0]], gbufs[kk], gsems[kk])
        for k in range(_SG):
            r = k % 2
            row = rowbase + k

            if k < 2:
                @pl.when(sg > 0)
                def _drain_prev_sg():
                    pltpu.make_async_copy(
                        y_hbm.at[pl.ds(0, _CKS)], sbufs[r], ssems[r]).wait()
            else:
                pltpu.make_async_copy(
                    y_hbm.at[pl.ds(0, _CKS)], sbufs[r], ssems[r]).wait()

            pltpu.make_async_copy(
                y_hbm.at[pl.ds(0, _CKS)], gbufs[r], gsems[r]).wait()

            def _sgrp(g16, carry2):
                off = jnp.minimum(g16 * 16, _CKS - 16)
                ew16 = ew_v[row, pl.ds(off, 16)]
                for jj in range(16):
                    w = ew16[jj]
                    for pz in range(_H // 16):
                        sl = pl.ds(pz * 16, 16)
                        sbufs[r][off + jj, sl] = gbufs[r][off + jj, sl] * w
                return carry2

            lax.fori_loop(0, _NG16, _sgrp, 0)

            if k == 2:
                @pl.when(sg < _NSGS - 1)
                def _fetch_next_record():
                    pltpu.async_copy(
                        rec_hbm.at[wid, sg + 1],
                        rec_v.at[pl.ds((1 - p) * _SG, _SG)],
                        rsem)
                    pltpu.async_copy(
                        ewr_hbm.at[wid, sg + 1],
                        ew_v.at[pl.ds((1 - p) * _SG, _SG)],
                        wsem)

            if k < _SG - 2:
                pltpu.async_copy(
                    y_hbm.at[rec_v.at[row + 2, 0]], gbufs[r], gsems[r])

            pltpu.async_copy(
                sbufs[r], acc_sh.at[rec_v.at[row, 1]], ssems[r], add=True)
        return carry

    lax.fori_loop(0, _NSGS, _sg_body, 0)
    for r in range(2):
        pltpu.make_async_copy(y_hbm.at[pl.ds(0, _CKS)], sbufs[r], ssems[r]).wait()
    plsc.subcore_barrier()
    pltpu.sync_copy(acc_sh.at[pl.ds(s * _RPT, _RPT)], out_hbm.at[c, pl.ds(s * _RPT, _RPT)])


def _sigmoid(v):
    return 1.0 / (1.0 + jnp.exp(-v))


def _tc_prep_body(deg0_ref, deg1_ref, dinv_ref):
    deg = deg0_ref[...] + deg1_ref[...] + 1.0
    safe = jnp.where(deg > 0, deg, 1.0)
    dinv_ref[...] = jnp.where(deg > 0, 1.0 / jnp.sqrt(safe), 0.0)


_tc_prep = pl.pallas_call(
    _tc_prep_body, out_shape=jax.ShapeDtypeStruct((_NPAD,), jnp.float32)
)


def _tc_y0_body(x_ref, w_ref, dinv_ref, y_ref):
    y_ref[...] = (
        jnp.dot(x_ref[...], w_ref[...], preferred_element_type=jnp.float32)
        * dinv_ref[...]
    )


_tc_y0 = pl.pallas_call(
    _tc_y0_body, out_shape=jax.ShapeDtypeStruct((_N, _H), jnp.float32)
)


def _bn_silu(p_ref, y_ref, dinv_ref, b_ref, g_ref, be_ref):
    pp = p_ref[0, :_N, :] + p_ref[1, :_N, :]
    dinv = dinv_ref[...]
    agg = dinv * (pp + y_ref[...]) + b_ref[...]
    mu = jnp.mean(agg, axis=0, keepdims=True)
    var = jnp.mean((agg - mu) ** 2, axis=0, keepdims=True)
    hn = (agg - mu) / jnp.sqrt(var + 1e-5) * g_ref[...] + be_ref[...]
    return hn * _sigmoid(hn)


def _tc_layer_body(p_ref, y_ref, dinv_ref, b_ref, g_ref, be_ref, w_ref, o_ref):
    h = _bn_silu(p_ref, y_ref, dinv_ref, b_ref, g_ref, be_ref)
    o_ref[...] = (
        jnp.dot(h, w_ref[...], preferred_element_type=jnp.float32) * dinv_ref[...]
    )


_tc_layer = pl.pallas_call(
    _tc_layer_body, out_shape=jax.ShapeDtypeStruct((_N, _H), jnp.float32)
)


def _tc_final_body(p_ref, y_ref, dinv_ref, b_ref, g_ref, be_ref,
                   fc1w_ref, fc1b_ref, fc2w_ref, fc2b_ref, o_ref):
    h = _bn_silu(p_ref, y_ref, dinv_ref, b_ref, g_ref, be_ref)
    pooled = jnp.mean(h, axis=0, keepdims=True)
    o1 = jnp.dot(pooled, fc1w_ref[...], preferred_element_type=jnp.float32) + fc1b_ref[...]
    o1 = o1 * _sigmoid(o1)
    o2 = jnp.dot(o1, fc2w_ref[...], preferred_element_type=jnp.float32) + fc2b_ref[...]
    o_ref[...] = _sigmoid(o2)


_tc_final = pl.pallas_call(
    _tc_final_body, out_shape=jax.ShapeDtypeStruct((1, 1), jnp.float32)
)


def kernel(x, edge_index, edge_attr, batch, W0, b0, g0, be0, W1, b1, g1, be1,
           W2, b2, g2, be2, W3, b3, g3, be3, fc1_W, fc1_b, fc2_W, fc2_b):
    src = edge_index[0].reshape(_NW, _NCHUNKS, _CKS)
    dst = edge_index[1].reshape(_NW, _NCHUNKS, _CKS)
    ew = edge_attr.reshape(_NW, _NCHUNKS, _CKS)
    rec = jnp.stack([src, dst], axis=2)
    rec = rec.reshape(_NW, _NSGS, _SG, 2, _CKS)
    ewr = ew.reshape(_NW, _NSGS, _SG, _CKS)

    deg0, deg1 = _sc_deg(edge_index[1], edge_attr.reshape(-1))
    dinv_full = _tc_prep(deg0, deg1)
    dinv_col = dinv_full[:_N, None]

    bs = (b0, b1, b2, b3)
    gs = (g0, g1, g2, g3)
    bes = (be0, be1, be2, be3)
    Ws = (W0, W1, W2, W3)

    y = _tc_y0(x, W0, dinv_col)
    out = None
    for i in range(4):
        parts = _sc_scatter(y, rec, ewr)
        if i < 3:
            y = _tc_layer(parts, y, dinv_col, bs[i], gs[i], bes[i], Ws[i + 1])
        else:
            out = _tc_final(parts, y, dinv_col, bs[3], gs[3], bes[3],
                            fc1_W, fc1_b, fc2_W, fc2_b)
    return out


# speedup vs baseline: 23.5683x; 2.0796x over previous
import functools

import jax
import jax.numpy as jnp
from jax import lax
from jax.experimental import pallas as pl
from jax.experimental.pallas import tpu as pltpu
from jax.experimental.pallas import tpu_sc as plsc

_N = 10000
_E = 640000
_H = 128
_NC = 2
_NS = 16
_NW = _NC * _NS
_EPW = _E // _NW
_CK = 80
_NCHUNK = _EPW // _CK
_NPAD = 10240
_RPT = _NPAD // _NS
_SG = 5
_NSG = _NCHUNK // _SG
_DRING = 5

_mesh = plsc.VectorSubcoreMesh(
    core_axis_name="c", subcore_axis_name="s", num_cores=_NC, num_subcores=_NS
)


_CKD = 100
_NCHUNKD = _EPW // _CKD


@functools.partial(
    pl.kernel,
    out_type=(jax.ShapeDtypeStruct((_NPAD,), jnp.float32),
              jax.ShapeDtypeStruct((_NPAD,), jnp.float32)),
    mesh=_mesh,
    scratch_types=[
        pltpu.VMEM((_NCHUNKD, _CKD), jnp.int32),
        pltpu.VMEM((_NCHUNKD, _CKD), jnp.float32),
        pltpu.VMEM((_RPT,), jnp.float32),
        pltpu.VMEM_SHARED((_NPAD,), jnp.float32),
    ],
)
def _sc_deg(dst_hbm, ew_hbm, out0_hbm, out1_hbm, dst2_v, ew2_v, zed_v, acc_sh):
    c = lax.axis_index("c")
    s = lax.axis_index("s")
    wid = s * _NC + c

    pltpu.sync_copy(dst_hbm.at[wid], dst2_v)
    pltpu.sync_copy(ew_hbm.at[wid], ew2_v)

    def _z(i, carry):
        zed_v[pl.ds(i * 16, 16)] = jnp.zeros((16,), jnp.float32)
        return carry

    lax.fori_loop(0, _RPT // 16, _z, 0)
    pltpu.sync_copy(zed_v, acc_sh.at[pl.ds(s * _RPT, _RPT)])
    plsc.subcore_barrier()

    def _chunk(i, carry):
        pltpu.sync_copy(ew2_v.at[i], acc_sh.at[dst2_v.at[i]], add=True)
        return carry

    lax.fori_loop(0, _NCHUNKD, _chunk, 0)
    plsc.subcore_barrier()

    @pl.when(c == 0)
    def _w0():
        pltpu.sync_copy(acc_sh.at[pl.ds(s * _RPT, _RPT)], out0_hbm.at[pl.ds(s * _RPT, _RPT)])

    @pl.when(c == 1)
    def _w1():
        pltpu.sync_copy(acc_sh.at[pl.ds(s * _RPT, _RPT)], out1_hbm.at[pl.ds(s * _RPT, _RPT)])


_CKS = 80
_NCHUNKS = _EPW // _CKS
_SGS = 6
_NSGS = 240 // _SGS
_NEPI = _NCHUNKS - _NSGS * _SGS
_NG16 = _CKS // 16


@functools.partial(
    pl.kernel,
    out_type=jax.ShapeDtypeStruct((_NC, _NPAD, _H), jnp.float32),
    mesh=_mesh,
    scratch_types=[
        pltpu.VMEM((16, 2, _CKS), jnp.int32),
        pltpu.VMEM((16, _CKS), jnp.float32),
    ]
    + [pltpu.VMEM((_CKS, _H), jnp.float32)] * 3
    + [pltpu.VMEM_SHARED((_NPAD, _H), jnp.float32)]
    + [pltpu.SemaphoreType.DMA] * 8,
)
def _sc_scatter(y_hbm, rec_hbm, ewr_hbm, repi_hbm, wepi_hbm, out_hbm, rec_v,
                ew_v, buf0, buf1, buf2, acc_sh, rsem, wsem,
                gsem0, gsem1, gsem2, ssem0, ssem1, ssem2):
    bufs = (buf0, buf1, buf2)
    gsems = (gsem0, gsem1, gsem2)
    ssems = (ssem0, ssem1, ssem2)

    c = lax.axis_index("c")
    s = lax.axis_index("s")
    wid = s * _NC + c

    pltpu.async_copy(rec_hbm.at[wid, 0], rec_v.at[pl.ds(0, 8)], rsem)
    pltpu.async_copy(ewr_hbm.at[wid, 0], ew_v.at[pl.ds(0, 8)], wsem)

    def _zrow(j, carry):
        for p in range(_H // 16):
            buf0[j, pl.ds(p * 16, 16)] = jnp.zeros((16,), jnp.float32)
        return carry

    lax.fori_loop(0, _CKS, _zrow, 0)
    for k in range(_RPT // _CKS):
        pltpu.sync_copy(buf0, acc_sh.at[pl.ds(s * _RPT + k * _CKS, _CKS)])

    pltpu.make_async_copy(
        rec_hbm.at[wid, 0], rec_v.at[pl.ds(0, 8)], rsem).wait()
    pltpu.make_async_copy(
        ewr_hbm.at[wid, 0], ew_v.at[pl.ds(0, 8)], wsem).wait()
    pltpu.async_copy(y_hbm.at[rec_v.at[0, 0]], buf0, gsems[0])
    plsc.subcore_barrier()

    def _scale(row, buf):
        def _sgrp(g16, carry2):
            off = g16 * 16
            ew16 = ew_v[row, pl.ds(off, 16)]
            for jj in range(16):
                w = ew16[jj]
                for pz in range(_H // 16):
                    sl = pl.ds(pz * 16, 16)
                    buf[off + jj, sl] = buf[off + jj, sl] * w
            return carry2

        lax.fori_loop(0, _NG16, _sgrp, 0)

    def _sg_body(sg, carry):
        p = lax.rem(sg, 2)
        rowbase = p * 8
        for k in range(_SGS):
            r = k % 3
            rn = (k + 1) % 3
            row = rowbase + k

            if k < 2:
                @pl.when(sg > 0)
                def _drain_prev_sg():
                    pltpu.make_async_copy(
                        y_hbm.at[pl.ds(0, _CKS)], bufs[rn], ssems[rn]).wait()
            else:
                pltpu.make_async_copy(
                    y_hbm.at[pl.ds(0, _CKS)], bufs[rn], ssems[rn]).wait()

            if k < _SGS - 1:
                pltpu.async_copy(
                    y_hbm.at[rec_v.at[row + 1, 0]], bufs[rn], gsems[rn])
            else:
                @pl.when(sg < _NSGS - 1)
                def _fire_next_sg():
                    pltpu.async_copy(
                        y_hbm.at[rec_v.at[(1 - p) * 8, 0]], bufs[rn], gsems[rn])

            pltpu.make_async_copy(
                y_hbm.at[pl.ds(0, _CKS)], bufs[r], gsems[r]).wait()

            _scale(row, bufs[r])

            if k == 2:
                @pl.when(sg < _NSGS - 1)
                def _fetch_next_record():
                    pltpu.async_copy(
                        rec_hbm.at[wid, sg + 1],
                        rec_v.at[pl.ds((1 - p) * 8, 8)], rsem)
                    pltpu.async_copy(
                        ewr_hbm.at[wid, sg + 1],
                        ew_v.at[pl.ds((1 - p) * 8, 8)], wsem)

            if k == 4:
                @pl.when(sg < _NSGS - 1)
                def _wait_next_record():
                    pltpu.make_async_copy(
                        rec_hbm.at[wid, 0],
                        rec_v.at[pl.ds(0, 8)], rsem).wait()
                    pltpu.make_async_copy(
                        ewr_hbm.at[wid, 0],
                        ew_v.at[pl.ds(0, 8)], wsem).wait()

            pltpu.async_copy(
                bufs[r], acc_sh.at[rec_v.at[row, 1]], ssems[r], add=True)
        return carry

    lax.fori_loop(0, _NSGS, _sg_body, 0)
    pltpu.make_async_copy(y_hbm.at[pl.ds(0, _CKS)], bufs[1], ssems[1]).wait()
    pltpu.make_async_copy(y_hbm.at[pl.ds(0, _CKS)], bufs[2], ssems[2]).wait()

    pltpu.sync_copy(repi_hbm.at[wid], rec_v.at[pl.ds(0, _NEPI)])
    pltpu.sync_copy(wepi_hbm.at[wid], ew_v.at[pl.ds(0, _NEPI)])

    pltpu.async_copy(y_hbm.at[rec_v.at[0, 0]], bufs[0], gsems[0])
    for j in range(_NEPI):
        r = j % 3
        rn = (j + 1) % 3
        if j >= 2:
            pltpu.make_async_copy(
                y_hbm.at[pl.ds(0, _CKS)], bufs[rn], ssems[rn]).wait()
        if j < _NEPI - 1:
            pltpu.async_copy(y_hbm.at[rec_v.at[j + 1, 0]], bufs[rn], gsems[rn])
        pltpu.make_async_copy(
            y_hbm.at[pl.ds(0, _CKS)], bufs[r], gsems[r]).wait()
        _scale(j, bufs[r])
        pltpu.async_copy(bufs[r], acc_sh.at[rec_v.at[j, 1]], ssems[r], add=True)
    pltpu.make_async_copy(
        y_hbm.at[pl.ds(0, _CKS)], bufs[(_NEPI - 2) % 3], ssems[(_NEPI - 2) % 3]).wait()
    pltpu.make_async_copy(
        y_hbm.at[pl.ds(0, _CKS)], bufs[(_NEPI - 1) % 3], ssems[(_NEPI - 1) % 3]).wait()
    plsc.subcore_barrier()
    pltpu.sync_copy(acc_sh.at[pl.ds(s * _RPT, _RPT)], out_hbm.at[c, pl.ds(s * _RPT, _RPT)])


def _sigmoid(v):
    return 1.0 / (1.0 + jnp.exp(-v))


def _tc_y0_body(deg0_ref, deg1_ref, x_ref, w_ref, y_ref, dinv_ref):
    deg = deg0_ref[...] + deg1_ref[...] + 1.0
    safe = jnp.where(deg > 0, deg, 1.0)
    dinv_ref[...] = jnp.where(deg > 0, 1.0 / jnp.sqrt(safe), 0.0)
    dcol = dinv_ref[: _N].reshape(_N, 1)
    y_ref[...] = (
        jnp.dot(x_ref[...], w_ref[...], preferred_element_type=jnp.float32)
        * dcol
    )


_tc_y0 = pl.pallas_call(
    _tc_y0_body,
    out_shape=(jax.ShapeDtypeStruct((_N, _H), jnp.float32),
               jax.ShapeDtypeStruct((_NPAD,), jnp.float32)),
)


def _bn_silu(p_ref, y_ref, dinv_ref, b_ref, g_ref, be_ref):
    pp = p_ref[0, :_N, :] + p_ref[1, :_N, :]
    dinv = dinv_ref[...]
    agg = dinv * (pp + y_ref[...]) + b_ref[...]
    mu = jnp.mean(agg, axis=0, keepdims=True)
    m2 = jnp.mean(agg * agg, axis=0, keepdims=True)
    var = m2 - mu * mu
    hn = (agg - mu) / jnp.sqrt(var + 1e-5) * g_ref[...] + be_ref[...]
    return hn * _sigmoid(hn)


def _tc_layer_body(p_ref, y_ref, dinv_ref, b_ref, g_ref, be_ref, w_ref, o_ref):
    h = _bn_silu(p_ref, y_ref, dinv_ref, b_ref, g_ref, be_ref)
    o_ref[...] = (
        jnp.dot(h, w_ref[...], preferred_element_type=jnp.float32) * dinv_ref[...]
    )


_tc_layer = pl.pallas_call(
    _tc_layer_body, out_shape=jax.ShapeDtypeStruct((_N, _H), jnp.float32)
)


def _tc_final_body(p_ref, y_ref, dinv_ref, b_ref, g_ref, be_ref,
                   fc1w_ref, fc1b_ref, fc2w_ref, fc2b_ref, o_ref):
    h = _bn_silu(p_ref, y_ref, dinv_ref, b_ref, g_ref, be_ref)
    pooled = jnp.mean(h, axis=0, keepdims=True)
    o1 = jnp.dot(pooled, fc1w_ref[...], preferred_element_type=jnp.float32) + fc1b_ref[...]
    o1 = o1 * _sigmoid(o1)
    o2 = jnp.dot(o1, fc2w_ref[...], preferred_element_type=jnp.float32) + fc2b_ref[...]
    o_ref[...] = _sigmoid(o2)


_tc_final = pl.pallas_call(
    _tc_final_body, out_shape=jax.ShapeDtypeStruct((1, 1), jnp.float32)
)


def kernel(x, edge_index, edge_attr, batch, W0, b0, g0, be0, W1, b1, g1, be1,
           W2, b2, g2, be2, W3, b3, g3, be3, fc1_W, fc1_b, fc2_W, fc2_b):
    srcf = edge_index[0].reshape(_NW, _NCHUNKS, _CKS)
    dstf = edge_index[1].reshape(_NW, _NCHUNKS, _CKS)
    ewf = edge_attr.reshape(_NW, _NCHUNKS, _CKS)
    nmain = _NSGS * _SGS
    rec = jnp.stack([srcf[:, :nmain], dstf[:, :nmain]], axis=2)
    rec = rec.reshape(_NW, _NSGS, _SGS, 2, _CKS)
    rec = jnp.pad(rec, ((0, 0), (0, 0), (0, 8 - _SGS), (0, 0), (0, 0)))
    ewr = ewf[:, :nmain].reshape(_NW, _NSGS, _SGS, _CKS)
    ewr = jnp.pad(ewr, ((0, 0), (0, 0), (0, 8 - _SGS), (0, 0)))
    repi = jnp.stack([srcf[:, nmain:], dstf[:, nmain:]], axis=2)
    wepi = ewf[:, nmain:]

    deg0, deg1 = _sc_deg(edge_index[1].reshape(_NW, _NCHUNKD, _CKD),
                         edge_attr.reshape(_NW, _NCHUNKD, _CKD))

    bs = (b0, b1, b2, b3)
    gs = (g0, g1, g2, g3)
    bes = (be0, be1, be2, be3)
    Ws = (W0, W1, W2, W3)

    y, dinv_full = _tc_y0(deg0, deg1, x, W0)
    dinv_col = dinv_full[:_N, None]
    out = None
    for i in range(4):
        parts = _sc_scatter(y, rec, ewr, repi, wepi)
        if i < 3:
            y = _tc_layer(parts, y, dinv_col, bs[i], gs[i], bes[i], Ws[i + 1])
        else:
            out = _tc_final(parts, y, dinv_col, bs[3], gs[3], bes[3],
                            fc1_W, fc1_b, fc2_W, fc2_b)
    return out
